# Initial kernel scaffold; baseline (speedup 1.0000x reference)
import functools

import jax
import jax.numpy as jnp
from jax import lax
from jax.experimental import pallas as pl
from jax.experimental.pallas import tpu as pltpu
from jax.experimental.pallas import tpu_sc as plsc

N_NODES = 10000
N_EDGES = 160000
DIM = 256
NC = 2
NS = 16
LANES = 16
DH = DIM // NC
EPT = N_EDGES // NS
CHUNK = 80
NCHUNK = EPT // CHUNK
ROWS_PT = N_NODES // NS


def _sc_aggregate(x0, x1, src_r, dst_r, z2d, z1d):
    mesh = plsc.VectorSubcoreMesh(
        core_axis_name="c", subcore_axis_name="s", num_cores=NC, num_subcores=NS
    )

    @functools.partial(
        pl.kernel,
        out_type=(
            jax.ShapeDtypeStruct((N_NODES, DH), jnp.float32),
            jax.ShapeDtypeStruct((N_NODES, DH), jnp.float32),
            jax.ShapeDtypeStruct((N_NODES,), jnp.float32),
        ),
        mesh=mesh,
        scratch_types=[
            pltpu.VMEM((NCHUNK, CHUNK), jnp.int32),
            pltpu.VMEM((NCHUNK, CHUNK), jnp.int32),
            pltpu.VMEM((CHUNK, DH), jnp.float32),
            pltpu.VMEM((CHUNK,), jnp.float32),
            pltpu.VMEM_SHARED((N_NODES, DH), jnp.float32),
            pltpu.VMEM_SHARED((N_NODES,), jnp.float32),
            pltpu.SemaphoreType.DMA,
        ],
    )
    def body(x0_hbm, x1_hbm, src_hbm, dst_hbm, z2d_hbm, z1d_hbm,
             agg0_hbm, agg1_hbm, deg_hbm,
             src_v, dst_v, rows_v, ones_v, acc_sh, deg_sh, sem):
        c = lax.axis_index("c")
        s = lax.axis_index("s")

        pltpu.sync_copy(z2d_hbm, acc_sh.at[pl.ds(s * ROWS_PT, ROWS_PT)])

        @pl.when(jnp.logical_and(c == 0, s == 0))
        def _():
            pltpu.sync_copy(z1d_hbm, deg_sh)

        pltpu.sync_copy(src_hbm.at[s], src_v)
        pltpu.sync_copy(dst_hbm.at[s], dst_v)
        for k in range(CHUNK // LANES):
            ones_v[pl.ds(k * LANES, LANES)] = jnp.ones((LANES,), jnp.float32)

        plsc.subcore_barrier()

        def chunk_step(j, _):
            @pl.when(c == 0)
            def _():
                pltpu.async_copy(x0_hbm.at[src_v.at[j]], rows_v, sem).wait()

            @pl.when(c == 1)
            def _():
                pltpu.async_copy(x1_hbm.at[src_v.at[j]], rows_v, sem).wait()

            pltpu.sync_copy(rows_v, acc_sh.at[dst_v.at[j]], add=True)

            @pl.when(c == 0)
            def _():
                pltpu.sync_copy(ones_v, deg_sh.at[dst_v.at[j]], add=True)

            return ()

        lax.fori_loop(0, NCHUNK, chunk_step, ())

        plsc.subcore_barrier()

        row0 = s * ROWS_PT

        @pl.when(c == 0)
        def _():
            pltpu.sync_copy(acc_sh.at[pl.ds(row0, ROWS_PT)],
                            agg0_hbm.at[pl.ds(row0, ROWS_PT)])

        @pl.when(c == 1)
        def _():
            pltpu.sync_copy(acc_sh.at[pl.ds(row0, ROWS_PT)],
                            agg1_hbm.at[pl.ds(row0, ROWS_PT)])

        @pl.when(jnp.logical_and(c == 0, s == 0))
        def _():
            pltpu.sync_copy(deg_sh, deg_hbm)

    return body(x0, x1, src_r, dst_r, z2d, z1d)


BN = 1000


def _tc_linear_body(a0_ref, a1_ref, deg_ref, w_ref, b_ref, o_ref):
    recip = 1.0 / jnp.maximum(deg_ref[...], 1.0)
    s0 = a0_ref[...] * recip
    s1 = a1_ref[...] * recip
    w = w_ref[...]
    acc = lax.dot_general(s0, w[:, :DH], (((1,), (1,)), ((), ())),
                          preferred_element_type=jnp.float32)
    acc += lax.dot_general(s1, w[:, DH:], (((1,), (1,)), ((), ())),
                           preferred_element_type=jnp.float32)
    o_ref[...] = acc + b_ref[...]


def _tc_linear(agg0, agg1, deg, W, b):
    grid = (N_NODES // BN,)
    return pl.pallas_call(
        _tc_linear_body,
        grid=grid,
        in_specs=[
            pl.BlockSpec((BN, DH), lambda i: (i, 0)),
            pl.BlockSpec((BN, DH), lambda i: (i, 0)),
            pl.BlockSpec((BN, 1), lambda i: (i, 0)),
            pl.BlockSpec((DIM, DIM), lambda i: (0, 0)),
            pl.BlockSpec((1, DIM), lambda i: (0, 0)),
        ],
        out_specs=pl.BlockSpec((BN, DIM), lambda i: (i, 0)),
        out_shape=jax.ShapeDtypeStruct((N_NODES, DIM), jnp.float32),
    )(agg0, agg1, deg, W, b)


def kernel(x, edge_index, W, b):
    ei = edge_index.astype(jnp.int32)
    dst = ei[0].reshape(NS, NCHUNK, CHUNK)
    src = ei[1].reshape(NS, NCHUNK, CHUNK)
    x0 = x[:, :DH]
    x1 = x[:, DH:]
    z2d = jnp.zeros((ROWS_PT, DH), jnp.float32)
    z1d = jnp.zeros((N_NODES,), jnp.float32)
    agg0, agg1, deg = _sc_aggregate(x0, x1, src, dst, z2d, z1d)
    return _tc_linear(agg0, agg1, deg.reshape(N_NODES, 1), W, b.reshape(1, DIM))


# speedup vs baseline: 5.6766x; 5.6766x over previous
import functools

import jax
import jax.numpy as jnp
from jax import lax
from jax.experimental import pallas as pl
from jax.experimental.pallas import tpu as pltpu
from jax.experimental.pallas import tpu_sc as plsc

N_NODES = 10000
N_EDGES = 160000
DIM = 256
NC = 2
NS = 16
LANES = 16
DH = DIM // NC
EPT = N_EDGES // NS
CHUNK = 80
NCHUNK = EPT // CHUNK
ROWS_PT = 1000
NW_ROWS = N_NODES // ROWS_PT


def _sc_aggregate(x0, x1, src_r, dst_r, z2d, z1d):
    mesh = plsc.VectorSubcoreMesh(
        core_axis_name="c", subcore_axis_name="s", num_cores=NC, num_subcores=NS
    )

    @functools.partial(
        pl.kernel,
        out_type=(
            jax.ShapeDtypeStruct((N_NODES, DH), jnp.float32),
            jax.ShapeDtypeStruct((N_NODES, DH), jnp.float32),
            jax.ShapeDtypeStruct((N_NODES,), jnp.float32),
        ),
        mesh=mesh,
        scratch_types=[
            pltpu.VMEM((NCHUNK, CHUNK), jnp.int32),
            pltpu.VMEM((NCHUNK, CHUNK), jnp.int32),
            pltpu.VMEM((CHUNK, DH), jnp.float32),
            pltpu.VMEM((CHUNK,), jnp.float32),
            pltpu.VMEM_SHARED((N_NODES, DH), jnp.float32),
            pltpu.VMEM_SHARED((N_NODES,), jnp.float32),
            pltpu.SemaphoreType.DMA,
        ],
    )
    def body(x0_hbm, x1_hbm, src_hbm, dst_hbm, z2d_hbm, z1d_hbm,
             agg0_hbm, agg1_hbm, deg_hbm,
             src_v, dst_v, rows_v, ones_v, acc_sh, deg_sh, sem):
        c = lax.axis_index("c")
        s = lax.axis_index("s")

        @pl.when(s < NW_ROWS)
        def _():
            pltpu.sync_copy(z2d_hbm, acc_sh.at[pl.ds(s * ROWS_PT, ROWS_PT)])

        @pl.when(jnp.logical_and(c == 0, s == 0))
        def _():
            pltpu.sync_copy(z1d_hbm, deg_sh)

        pltpu.sync_copy(src_hbm.at[s], src_v)
        pltpu.sync_copy(dst_hbm.at[s], dst_v)
        for k in range(CHUNK // LANES):
            ones_v[pl.ds(k * LANES, LANES)] = jnp.ones((LANES,), jnp.float32)

        plsc.subcore_barrier()

        def chunk_step(j, _):
            @pl.when(c == 0)
            def _():
                pltpu.async_copy(x0_hbm.at[src_v.at[j]], rows_v, sem).wait()

            @pl.when(c == 1)
            def _():
                pltpu.async_copy(x1_hbm.at[src_v.at[j]], rows_v, sem).wait()

            pltpu.sync_copy(rows_v, acc_sh.at[dst_v.at[j]], add=True)

            @pl.when(c == 0)
            def _():
                pltpu.sync_copy(ones_v, deg_sh.at[dst_v.at[j]], add=True)

            return ()

        lax.fori_loop(0, NCHUNK, chunk_step, ())

        plsc.subcore_barrier()

        row0 = s * ROWS_PT

        @pl.when(jnp.logical_and(c == 0, s < NW_ROWS))
        def _():
            pltpu.sync_copy(acc_sh.at[pl.ds(row0, ROWS_PT)],
                            agg0_hbm.at[pl.ds(row0, ROWS_PT)])

        @pl.when(jnp.logical_and(c == 1, s < NW_ROWS))
        def _():
            pltpu.sync_copy(acc_sh.at[pl.ds(row0, ROWS_PT)],
                            agg1_hbm.at[pl.ds(row0, ROWS_PT)])

        @pl.when(jnp.logical_and(c == 0, s == 0))
        def _():
            pltpu.sync_copy(deg_sh, deg_hbm)

    return body(x0, x1, src_r, dst_r, z2d, z1d)


BN = 1000


def _tc_linear_body(a0_ref, a1_ref, deg_ref, w_ref, b_ref, o_ref):
    recip = 1.0 / jnp.maximum(deg_ref[...], 1.0)
    s0 = a0_ref[...] * recip
    s1 = a1_ref[...] * recip
    w = w_ref[...]
    acc = lax.dot_general(s0, w[:, :DH], (((1,), (1,)), ((), ())),
                          preferred_element_type=jnp.float32)
    acc += lax.dot_general(s1, w[:, DH:], (((1,), (1,)), ((), ())),
                           preferred_element_type=jnp.float32)
    o_ref[...] = acc + b_ref[...]


def _tc_linear(agg0, agg1, deg, W, b):
    grid = (N_NODES // BN,)
    return pl.pallas_call(
        _tc_linear_body,
        grid=grid,
        in_specs=[
            pl.BlockSpec((BN, DH), lambda i: (i, 0)),
            pl.BlockSpec((BN, DH), lambda i: (i, 0)),
            pl.BlockSpec((BN, 1), lambda i: (i, 0)),
            pl.BlockSpec((DIM, DIM), lambda i: (0, 0)),
            pl.BlockSpec((1, DIM), lambda i: (0, 0)),
        ],
        out_specs=pl.BlockSpec((BN, DIM), lambda i: (i, 0)),
        out_shape=jax.ShapeDtypeStruct((N_NODES, DIM), jnp.float32),
    )(agg0, agg1, deg, W, b)


def kernel(x, edge_index, W, b):
    ei = edge_index.astype(jnp.int32)
    dst = ei[0].reshape(NS, NCHUNK, CHUNK)
    src = ei[1].reshape(NS, NCHUNK, CHUNK)
    x0 = x[:, :DH]
    x1 = x[:, DH:]
    z2d = jnp.zeros((ROWS_PT, DH), jnp.float32)
    z1d = jnp.zeros((N_NODES,), jnp.float32)
    agg0, agg1, deg = _sc_aggregate(x0, x1, src, dst, z2d, z1d)
    return _tc_linear(agg0, agg1, deg.reshape(N_NODES, 1), W, b.reshape(1, DIM))


# speedup vs baseline: 8.5774x; 1.5110x over previous
import functools

import jax
import jax.numpy as jnp
from jax import lax
from jax.experimental import pallas as pl
from jax.experimental.pallas import tpu as pltpu
from jax.experimental.pallas import tpu_sc as plsc

N_NODES = 10000
N_EDGES = 160000
DIM = 256
NC = 2
NS = 16
LANES = 16
DH = DIM // NC
EPT = N_EDGES // NS
CHUNK = 100
NCHUNK = EPT // CHUNK
NHALF = 2
NCHUNK_H = NCHUNK // NHALF
NPAIR_H = NCHUNK_H // 2
ROWS_PT = 1000
NW_ROWS = N_NODES // ROWS_PT


def _sc_aggregate(x0, x1, src_r, dst_r, z2d, z1d, o1):
    mesh = plsc.VectorSubcoreMesh(
        core_axis_name="c", subcore_axis_name="s", num_cores=NC, num_subcores=NS
    )

    @functools.partial(
        pl.kernel,
        out_type=(
            jax.ShapeDtypeStruct((N_NODES, DH), jnp.float32),
            jax.ShapeDtypeStruct((N_NODES, DH), jnp.float32),
            jax.ShapeDtypeStruct((N_NODES,), jnp.float32),
            jax.ShapeDtypeStruct((N_NODES,), jnp.float32),
        ),
        mesh=mesh,
        scratch_types=[
            pltpu.VMEM((NCHUNK_H, CHUNK), jnp.int32),
            pltpu.VMEM((NCHUNK_H, CHUNK), jnp.int32),
            pltpu.VMEM((CHUNK, DH), jnp.float32),
            pltpu.VMEM((CHUNK, DH), jnp.float32),
            pltpu.VMEM((CHUNK,), jnp.float32),
            pltpu.VMEM_SHARED((N_NODES, DH), jnp.float32),
            pltpu.VMEM_SHARED((N_NODES,), jnp.float32),
            pltpu.SemaphoreType.DMA,
            pltpu.SemaphoreType.DMA,
        ],
    )
    def body(x0_hbm, x1_hbm, src_hbm, dst_hbm, z2d_hbm, z1d_hbm, o1_hbm,
             agg0_hbm, agg1_hbm, deg0_hbm, deg1_hbm,
             src_v, dst_v, rows_a, rows_b, ones_v, acc_sh, deg_sh,
             sem_a, sem_b):
        c = lax.axis_index("c")
        s = lax.axis_index("s")

        @pl.when(s < NW_ROWS)
        def _():
            pltpu.sync_copy(z2d_hbm, acc_sh.at[pl.ds(s * ROWS_PT, ROWS_PT)])

        @pl.when(s == 0)
        def _():
            pltpu.sync_copy(z1d_hbm, deg_sh)

        pltpu.sync_copy(o1_hbm, ones_v)

        plsc.subcore_barrier()

        def gissue(j, buf, sem):
            @pl.when(c == 0)
            def _():
                pltpu.async_copy(x0_hbm.at[src_v.at[j]], buf, sem)

            @pl.when(c == 1)
            def _():
                pltpu.async_copy(x1_hbm.at[src_v.at[j]], buf, sem)

        def gwait(j, buf, sem):
            @pl.when(c == 0)
            def _():
                pltpu.make_async_copy(x0_hbm.at[src_v.at[j]], buf, sem).wait()

            @pl.when(c == 1)
            def _():
                pltpu.make_async_copy(x1_hbm.at[src_v.at[j]], buf, sem).wait()

        def pair_step(jp, _):
            j0 = 2 * jp
            j1 = j0 + 1
            gissue(j1, rows_b, sem_b)
            gwait(j0, rows_a, sem_a)
            pltpu.sync_copy(rows_a, acc_sh.at[dst_v.at[j0]], add=True)

            @pl.when(c == 0)
            def _():
                pltpu.sync_copy(ones_v, deg_sh.at[dst_v.at[j0]], add=True)

            @pl.when(jp + 1 < NPAIR_H)
            def _():
                gissue(j0 + 2, rows_a, sem_a)

            gwait(j1, rows_b, sem_b)
            pltpu.sync_copy(rows_b, acc_sh.at[dst_v.at[j1]], add=True)

            @pl.when(c == 1)
            def _():
                pltpu.sync_copy(ones_v, deg_sh.at[dst_v.at[j1]], add=True)

            return ()

        for h in range(NHALF):
            pltpu.sync_copy(src_hbm.at[h].at[s], src_v)
            pltpu.sync_copy(dst_hbm.at[h].at[s], dst_v)
            gissue(0, rows_a, sem_a)
            lax.fori_loop(0, NPAIR_H, pair_step, ())

        plsc.subcore_barrier()

        row0 = s * ROWS_PT

        @pl.when(jnp.logical_and(c == 0, s < NW_ROWS))
        def _():
            pltpu.sync_copy(acc_sh.at[pl.ds(row0, ROWS_PT)],
                            agg0_hbm.at[pl.ds(row0, ROWS_PT)])

        @pl.when(jnp.logical_and(c == 1, s < NW_ROWS))
        def _():
            pltpu.sync_copy(acc_sh.at[pl.ds(row0, ROWS_PT)],
                            agg1_hbm.at[pl.ds(row0, ROWS_PT)])

        @pl.when(jnp.logical_and(c == 0, s == NS - 1))
        def _():
            pltpu.sync_copy(deg_sh, deg0_hbm)

        @pl.when(jnp.logical_and(c == 1, s == NS - 1))
        def _():
            pltpu.sync_copy(deg_sh, deg1_hbm)

    return body(x0, x1, src_r, dst_r, z2d, z1d, o1)


BN = 1000


def _tc_linear_body(a0_ref, a1_ref, d0_ref, d1_ref, w_ref, b_ref, o_ref):
    deg = d0_ref[...] + d1_ref[...]
    recip = 1.0 / jnp.maximum(deg, 1.0)
    s0 = a0_ref[...] * recip
    s1 = a1_ref[...] * recip
    w = w_ref[...]
    acc = lax.dot_general(s0, w[:, :DH], (((1,), (1,)), ((), ())),
                          preferred_element_type=jnp.float32)
    acc += lax.dot_general(s1, w[:, DH:], (((1,), (1,)), ((), ())),
                           preferred_element_type=jnp.float32)
    o_ref[...] = acc + b_ref[...]


def _tc_linear(agg0, agg1, deg0, deg1, W, b):
    grid = (N_NODES // BN,)
    return pl.pallas_call(
        _tc_linear_body,
        grid=grid,
        in_specs=[
            pl.BlockSpec((BN, DH), lambda i: (i, 0)),
            pl.BlockSpec((BN, DH), lambda i: (i, 0)),
            pl.BlockSpec((BN, 1), lambda i: (i, 0)),
            pl.BlockSpec((BN, 1), lambda i: (i, 0)),
            pl.BlockSpec((DIM, DIM), lambda i: (0, 0)),
            pl.BlockSpec((1, DIM), lambda i: (0, 0)),
        ],
        out_specs=pl.BlockSpec((BN, DIM), lambda i: (i, 0)),
        out_shape=jax.ShapeDtypeStruct((N_NODES, DIM), jnp.float32),
    )(agg0, agg1, deg0, deg1, W, b)


def kernel(x, edge_index, W, b):
    ei = edge_index.astype(jnp.int32)
    dst = ei[0].reshape(NS, NHALF, NCHUNK_H, CHUNK).swapaxes(0, 1)
    src = ei[1].reshape(NS, NHALF, NCHUNK_H, CHUNK).swapaxes(0, 1)
    x0 = x[:, :DH]
    x1 = x[:, DH:]
    z2d = jnp.zeros((ROWS_PT, DH), jnp.float32)
    z1d = jnp.zeros((N_NODES,), jnp.float32)
    o1 = jnp.ones((CHUNK,), jnp.float32)
    agg0, agg1, deg0, deg1 = _sc_aggregate(x0, x1, src, dst, z2d, z1d, o1)
    return _tc_linear(agg0, agg1, deg0.reshape(N_NODES, 1),
                      deg1.reshape(N_NODES, 1), W, b.reshape(1, DIM))


# speedup vs baseline: 8.6460x; 1.0080x over previous
import functools

import jax
import jax.numpy as jnp
from jax import lax
from jax.experimental import pallas as pl
from jax.experimental.pallas import tpu as pltpu
from jax.experimental.pallas import tpu_sc as plsc

N_NODES = 10000
N_EDGES = 160000
DIM = 256
NC = 2
NS = 16
LANES = 16
DH = DIM // NC
EPT = N_EDGES // NS
CHUNK = 100
NCHUNK = EPT // CHUNK
NHALF = 2
NCHUNK_H = NCHUNK // NHALF
NPAIR_H = NCHUNK_H // 2
ROWS_PT = 1000
NW_ROWS = N_NODES // ROWS_PT


def _sc_aggregate(x2, src0_r, src1_r, dst_r, z2d, z1d, o1):
    mesh = plsc.VectorSubcoreMesh(
        core_axis_name="c", subcore_axis_name="s", num_cores=NC, num_subcores=NS
    )

    @functools.partial(
        pl.kernel,
        out_type=(
            jax.ShapeDtypeStruct((N_NODES, DH), jnp.float32),
            jax.ShapeDtypeStruct((N_NODES, DH), jnp.float32),
            jax.ShapeDtypeStruct((N_NODES,), jnp.float32),
            jax.ShapeDtypeStruct((N_NODES,), jnp.float32),
        ),
        mesh=mesh,
        scratch_types=[
            pltpu.VMEM((NCHUNK_H, CHUNK), jnp.int32),
            pltpu.VMEM((NCHUNK_H, CHUNK), jnp.int32),
            pltpu.VMEM((CHUNK, DH), jnp.float32),
            pltpu.VMEM((CHUNK, DH), jnp.float32),
            pltpu.VMEM((CHUNK,), jnp.float32),
            pltpu.VMEM_SHARED((N_NODES, DH), jnp.float32),
            pltpu.VMEM_SHARED((N_NODES,), jnp.float32),
            pltpu.SemaphoreType.DMA,
            pltpu.SemaphoreType.DMA,
            pltpu.SemaphoreType.DMA,
        ],
    )
    def body(x2_hbm, src0_hbm, src1_hbm, dst_hbm, z2d_hbm, z1d_hbm, o1_hbm,
             agg0_hbm, agg1_hbm, deg0_hbm, deg1_hbm,
             src_v, dst_v, rows_a, rows_b, ones_v, acc_sh, deg_sh,
             sem_a, sem_b, sem_d):
        c = lax.axis_index("c")
        s = lax.axis_index("s")

        @pl.when(s < NW_ROWS)
        def _():
            pltpu.sync_copy(z2d_hbm, acc_sh.at[pl.ds(s * ROWS_PT, ROWS_PT)])

        @pl.when(s == 0)
        def _():
            pltpu.sync_copy(z1d_hbm, deg_sh)

        pltpu.sync_copy(o1_hbm, ones_v)

        plsc.subcore_barrier()

        def gissue(j, buf, sem):
            pltpu.async_copy(x2_hbm.at[src_v.at[j]], buf, sem)

        def gwait(j, buf, sem):
            pltpu.make_async_copy(x2_hbm.at[src_v.at[j]], buf, sem).wait()

        def pair_step(jp, _):
            j0 = 2 * jp
            j1 = j0 + 1
            gissue(j1, rows_b, sem_b)
            gwait(j0, rows_a, sem_a)
            pltpu.sync_copy(rows_a, acc_sh.at[dst_v.at[j0]], add=True)

            @pl.when(c == 0)
            def _():
                pltpu.async_copy(ones_v, deg_sh.at[dst_v.at[j0]], sem_d,
                                 add=True)

            @pl.when(jp + 1 < NPAIR_H)
            def _():
                gissue(j0 + 2, rows_a, sem_a)

            gwait(j1, rows_b, sem_b)
            pltpu.sync_copy(rows_b, acc_sh.at[dst_v.at[j1]], add=True)

            @pl.when(c == 1)
            def _():
                pltpu.async_copy(ones_v, deg_sh.at[dst_v.at[j1]], sem_d,
                                 add=True)

            return ()

        def deg_drain(i, _):
            pltpu.make_async_copy(ones_v, deg_sh.at[dst_v.at[0]],
                                  sem_d).wait()
            return ()

        for h in range(NHALF):
            @pl.when(c == 0)
            def _():
                pltpu.sync_copy(src0_hbm.at[h].at[s], src_v)

            @pl.when(c == 1)
            def _():
                pltpu.sync_copy(src1_hbm.at[h].at[s], src_v)

            pltpu.sync_copy(dst_hbm.at[h].at[s], dst_v)
            gissue(0, rows_a, sem_a)
            lax.fori_loop(0, NPAIR_H, pair_step, ())
            lax.fori_loop(0, NPAIR_H, deg_drain, ())

        plsc.subcore_barrier()

        row0 = s * ROWS_PT

        @pl.when(jnp.logical_and(c == 0, s < NW_ROWS))
        def _():
            pltpu.sync_copy(acc_sh.at[pl.ds(row0, ROWS_PT)],
                            agg0_hbm.at[pl.ds(row0, ROWS_PT)])

        @pl.when(jnp.logical_and(c == 1, s < NW_ROWS))
        def _():
            pltpu.sync_copy(acc_sh.at[pl.ds(row0, ROWS_PT)],
                            agg1_hbm.at[pl.ds(row0, ROWS_PT)])

        @pl.when(jnp.logical_and(c == 0, s == NS - 1))
        def _():
            pltpu.sync_copy(deg_sh, deg0_hbm)

        @pl.when(jnp.logical_and(c == 1, s == NS - 1))
        def _():
            pltpu.sync_copy(deg_sh, deg1_hbm)

    return body(x2, src0_r, src1_r, dst_r, z2d, z1d, o1)


BN = 2000


def _tc_linear_body(a0_ref, a1_ref, d0_ref, d1_ref, w_ref, b_ref, o_ref):
    deg = d0_ref[...] + d1_ref[...]
    recip = 1.0 / jnp.maximum(deg, 1.0)
    s0 = a0_ref[...] * recip
    s1 = a1_ref[...] * recip
    w = w_ref[...]
    acc = lax.dot_general(s0, w[:, :DH], (((1,), (1,)), ((), ())),
                          preferred_element_type=jnp.float32)
    acc += lax.dot_general(s1, w[:, DH:], (((1,), (1,)), ((), ())),
                           preferred_element_type=jnp.float32)
    o_ref[...] = acc + b_ref[...]


def _tc_linear(agg0, agg1, deg0, deg1, W, b):
    grid = (N_NODES // BN,)
    return pl.pallas_call(
        _tc_linear_body,
        grid=grid,
        in_specs=[
            pl.BlockSpec((BN, DH), lambda i: (i, 0)),
            pl.BlockSpec((BN, DH), lambda i: (i, 0)),
            pl.BlockSpec((BN, 1), lambda i: (i, 0)),
            pl.BlockSpec((BN, 1), lambda i: (i, 0)),
            pl.BlockSpec((DIM, DIM), lambda i: (0, 0)),
            pl.BlockSpec((1, DIM), lambda i: (0, 0)),
        ],
        out_specs=pl.BlockSpec((BN, DIM), lambda i: (i, 0)),
        out_shape=jax.ShapeDtypeStruct((N_NODES, DIM), jnp.float32),
    )(agg0, agg1, deg0, deg1, W, b)


def kernel(x, edge_index, W, b):
    ei = edge_index.astype(jnp.int32)
    dst = ei[0].reshape(NS, NHALF, NCHUNK_H, CHUNK).swapaxes(0, 1)
    src2 = ei[1] * 2
    src0 = src2.reshape(NS, NHALF, NCHUNK_H, CHUNK).swapaxes(0, 1)
    src1 = (src2 + 1).reshape(NS, NHALF, NCHUNK_H, CHUNK).swapaxes(0, 1)
    x2 = x.reshape(2 * N_NODES, DH)
    z2d = jnp.zeros((ROWS_PT, DH), jnp.float32)
    z1d = jnp.zeros((N_NODES,), jnp.float32)
    o1 = jnp.ones((CHUNK,), jnp.float32)
    agg0, agg1, deg0, deg1 = _sc_aggregate(x2, src0, src1, dst, z2d, z1d, o1)
    return _tc_linear(agg0, agg1, deg0.reshape(N_NODES, 1),
                      deg1.reshape(N_NODES, 1), W, b.reshape(1, DIM))


# speedup vs baseline: 8.8044x; 1.0183x over previous
import functools

import jax
import jax.numpy as jnp
from jax import lax
from jax.experimental import pallas as pl
from jax.experimental.pallas import tpu as pltpu
from jax.experimental.pallas import tpu_sc as plsc

N_NODES = 10000
N_EDGES = 160000
DIM = 256
NC = 2
NS = 16
LANES = 16
DH = DIM // NC
EPT = N_EDGES // NS
CHUNK = 125
NCHUNK = EPT // CHUNK
NHALF = 2
NCHUNK_H = NCHUNK // NHALF
NPAIR_H = NCHUNK_H // 2
ROWS_PT = 1000
NW_ROWS = N_NODES // ROWS_PT


def _sc_aggregate(x2, src0_r, src1_r, dst_r, z2d, z1d, o1):
    mesh = plsc.VectorSubcoreMesh(
        core_axis_name="c", subcore_axis_name="s", num_cores=NC, num_subcores=NS
    )

    @functools.partial(
        pl.kernel,
        out_type=(
            jax.ShapeDtypeStruct((N_NODES, DH), jnp.float32),
            jax.ShapeDtypeStruct((N_NODES, DH), jnp.float32),
            jax.ShapeDtypeStruct((N_NODES,), jnp.float32),
            jax.ShapeDtypeStruct((N_NODES,), jnp.float32),
        ),
        mesh=mesh,
        scratch_types=[
            pltpu.VMEM((NCHUNK_H, CHUNK), jnp.int32),
            pltpu.VMEM((NCHUNK_H, CHUNK), jnp.int32),
            pltpu.VMEM((CHUNK, DH), jnp.float32),
            pltpu.VMEM((CHUNK, DH), jnp.float32),
            pltpu.VMEM((CHUNK,), jnp.float32),
            pltpu.VMEM_SHARED((N_NODES, DH), jnp.float32),
            pltpu.VMEM_SHARED((N_NODES,), jnp.float32),
            pltpu.SemaphoreType.DMA,
            pltpu.SemaphoreType.DMA,
            pltpu.SemaphoreType.DMA,
        ],
    )
    def body(x2_hbm, src0_hbm, src1_hbm, dst_hbm, z2d_hbm, z1d_hbm, o1_hbm,
             agg0_hbm, agg1_hbm, deg0_hbm, deg1_hbm,
             src_v, dst_v, rows_a, rows_b, ones_v, acc_sh, deg_sh,
             sem_a, sem_b, sem_d):
        c = lax.axis_index("c")
        s = lax.axis_index("s")

        @pl.when(s < NW_ROWS)
        def _():
            pltpu.sync_copy(z2d_hbm, acc_sh.at[pl.ds(s * ROWS_PT, ROWS_PT)])

        @pl.when(s == 0)
        def _():
            pltpu.sync_copy(z1d_hbm, deg_sh)

        pltpu.sync_copy(o1_hbm, ones_v)

        plsc.subcore_barrier()

        def gissue(j, buf, sem):
            pltpu.async_copy(x2_hbm.at[src_v.at[j]], buf, sem)

        def gwait(j, buf, sem):
            pltpu.make_async_copy(x2_hbm.at[src_v.at[j]], buf, sem).wait()

        def pair_step(jp, _):
            j0 = 2 * jp
            j1 = j0 + 1
            gissue(j1, rows_b, sem_b)
            gwait(j0, rows_a, sem_a)
            pltpu.sync_copy(rows_a, acc_sh.at[dst_v.at[j0]], add=True)

            @pl.when(c == 0)
            def _():
                pltpu.async_copy(ones_v, deg_sh.at[dst_v.at[j0]], sem_d,
                                 add=True)

            @pl.when(jp + 1 < NPAIR_H)
            def _():
                gissue(j0 + 2, rows_a, sem_a)

            gwait(j1, rows_b, sem_b)
            pltpu.sync_copy(rows_b, acc_sh.at[dst_v.at[j1]], add=True)

            @pl.when(c == 1)
            def _():
                pltpu.async_copy(ones_v, deg_sh.at[dst_v.at[j1]], sem_d,
                                 add=True)

            return ()

        def deg_drain(i, _):
            pltpu.make_async_copy(ones_v, deg_sh.at[dst_v.at[0]],
                                  sem_d).wait()
            return ()

        for h in range(NHALF):
            @pl.when(c == 0)
            def _():
                pltpu.sync_copy(src0_hbm.at[h].at[s], src_v)

            @pl.when(c == 1)
            def _():
                pltpu.sync_copy(src1_hbm.at[h].at[s], src_v)

            pltpu.sync_copy(dst_hbm.at[h].at[s], dst_v)
            gissue(0, rows_a, sem_a)
            lax.fori_loop(0, NPAIR_H, pair_step, ())
            lax.fori_loop(0, NPAIR_H, deg_drain, ())

        plsc.subcore_barrier()

        row0 = s * ROWS_PT

        @pl.when(jnp.logical_and(c == 0, s < NW_ROWS))
        def _():
            pltpu.sync_copy(acc_sh.at[pl.ds(row0, ROWS_PT)],
                            agg0_hbm.at[pl.ds(row0, ROWS_PT)])

        @pl.when(jnp.logical_and(c == 1, s < NW_ROWS))
        def _():
            pltpu.sync_copy(acc_sh.at[pl.ds(row0, ROWS_PT)],
                            agg1_hbm.at[pl.ds(row0, ROWS_PT)])

        @pl.when(jnp.logical_and(c == 0, s == NS - 1))
        def _():
            pltpu.sync_copy(deg_sh, deg0_hbm)

        @pl.when(jnp.logical_and(c == 1, s == NS - 1))
        def _():
            pltpu.sync_copy(deg_sh, deg1_hbm)

    return body(x2, src0_r, src1_r, dst_r, z2d, z1d, o1)


BN = 2000


def _tc_linear_body(a0_ref, a1_ref, d0_ref, d1_ref, w_ref, b_ref, o_ref):
    deg = d0_ref[...] + d1_ref[...]
    recip = 1.0 / jnp.maximum(deg, 1.0)
    s0 = a0_ref[...] * recip
    s1 = a1_ref[...] * recip
    w = w_ref[...]
    acc = lax.dot_general(s0, w[:, :DH], (((1,), (1,)), ((), ())),
                          preferred_element_type=jnp.float32)
    acc += lax.dot_general(s1, w[:, DH:], (((1,), (1,)), ((), ())),
                           preferred_element_type=jnp.float32)
    o_ref[...] = acc + b_ref[...]


def _tc_linear(agg0, agg1, deg0, deg1, W, b):
    grid = (N_NODES // BN,)
    return pl.pallas_call(
        _tc_linear_body,
        grid=grid,
        in_specs=[
            pl.BlockSpec((BN, DH), lambda i: (i, 0)),
            pl.BlockSpec((BN, DH), lambda i: (i, 0)),
            pl.BlockSpec((BN, 1), lambda i: (i, 0)),
            pl.BlockSpec((BN, 1), lambda i: (i, 0)),
            pl.BlockSpec((DIM, DIM), lambda i: (0, 0)),
            pl.BlockSpec((1, DIM), lambda i: (0, 0)),
        ],
        out_specs=pl.BlockSpec((BN, DIM), lambda i: (i, 0)),
        out_shape=jax.ShapeDtypeStruct((N_NODES, DIM), jnp.float32),
    )(agg0, agg1, deg0, deg1, W, b)


def kernel(x, edge_index, W, b):
    ei = edge_index.astype(jnp.int32)
    dst = ei[0].reshape(NS, NHALF, NCHUNK_H, CHUNK).swapaxes(0, 1)
    src2 = ei[1] * 2
    src0 = src2.reshape(NS, NHALF, NCHUNK_H, CHUNK).swapaxes(0, 1)
    src1 = (src2 + 1).reshape(NS, NHALF, NCHUNK_H, CHUNK).swapaxes(0, 1)
    x2 = x.reshape(2 * N_NODES, DH)
    z2d = jnp.zeros((ROWS_PT, DH), jnp.float32)
    z1d = jnp.zeros((N_NODES,), jnp.float32)
    o1 = jnp.ones((CHUNK,), jnp.float32)
    agg0, agg1, deg0, deg1 = _sc_aggregate(x2, src0, src1, dst, z2d, z1d, o1)
    return _tc_linear(agg0, agg1, deg0.reshape(N_NODES, 1),
                      deg1.reshape(N_NODES, 1), W, b.reshape(1, DIM))


# speedup vs baseline: 9.0665x; 1.0298x over previous
import functools

import jax
import jax.numpy as jnp
from jax import lax
from jax.experimental import pallas as pl
from jax.experimental.pallas import tpu as pltpu
from jax.experimental.pallas import tpu_sc as plsc

N_NODES = 10000
N_EDGES = 160000
DIM = 256
NC = 2
NS = 16
LANES = 16
DH = DIM // NC
EPT = N_EDGES // NS
CHUNK = 125
NCHUNK = EPT // CHUNK
NHALF = 2
NCHUNK_H = NCHUNK // NHALF
NPAIR_H = NCHUNK_H // 2
ROWS_PT = 1000
NW_ROWS = N_NODES // ROWS_PT


def _sc_aggregate(x2, src0_r, src1_r, dst_r, z2d, z1d, o1):
    mesh = plsc.VectorSubcoreMesh(
        core_axis_name="c", subcore_axis_name="s", num_cores=NC, num_subcores=NS
    )

    @functools.partial(
        pl.kernel,
        out_type=(
            jax.ShapeDtypeStruct((N_NODES, DH), jnp.float32),
            jax.ShapeDtypeStruct((N_NODES, DH), jnp.float32),
            jax.ShapeDtypeStruct((N_NODES,), jnp.float32),
            jax.ShapeDtypeStruct((N_NODES,), jnp.float32),
        ),
        mesh=mesh,
        scratch_types=[
            pltpu.VMEM((NCHUNK_H, CHUNK), jnp.int32),
            pltpu.VMEM((NCHUNK_H, CHUNK), jnp.int32),
            pltpu.VMEM((CHUNK, DH), jnp.float32),
            pltpu.VMEM((CHUNK, DH), jnp.float32),
            pltpu.VMEM((CHUNK,), jnp.float32),
            pltpu.VMEM_SHARED((N_NODES, DH), jnp.float32),
            pltpu.VMEM_SHARED((N_NODES,), jnp.float32),
            pltpu.SemaphoreType.DMA,
            pltpu.SemaphoreType.DMA,
            pltpu.SemaphoreType.DMA,
        ],
    )
    def body(x2_hbm, src0_hbm, src1_hbm, dst_hbm, z2d_hbm, z1d_hbm, o1_hbm,
             agg0_hbm, agg1_hbm, deg0_hbm, deg1_hbm,
             src_v, dst_v, rows_a, rows_b, ones_v, acc_sh, deg_sh,
             sem_a, sem_b, sem_d):
        c = lax.axis_index("c")
        s = lax.axis_index("s")

        @pl.when(s < NW_ROWS)
        def _():
            pltpu.sync_copy(z2d_hbm, acc_sh.at[pl.ds(s * ROWS_PT, ROWS_PT)])

        @pl.when(s == 0)
        def _():
            pltpu.sync_copy(z1d_hbm, deg_sh)

        pltpu.sync_copy(o1_hbm, ones_v)

        plsc.subcore_barrier()

        def gissue(j, buf, sem):
            pltpu.async_copy(x2_hbm.at[src_v.at[j]], buf, sem)

        def gwait(j, buf, sem):
            pltpu.make_async_copy(x2_hbm.at[src_v.at[j]], buf, sem).wait()

        def pair_step(jp, _):
            j0 = 2 * jp
            j1 = j0 + 1
            gissue(j1, rows_b, sem_b)
            gwait(j0, rows_a, sem_a)
            pltpu.sync_copy(rows_a, acc_sh.at[dst_v.at[j0]], add=True)

            @pl.when(c == 0)
            def _():
                pltpu.async_copy(ones_v, deg_sh.at[dst_v.at[j0]], sem_d,
                                 add=True)

            @pl.when(jp + 1 < NPAIR_H)
            def _():
                gissue(j0 + 2, rows_a, sem_a)

            gwait(j1, rows_b, sem_b)
            pltpu.sync_copy(rows_b, acc_sh.at[dst_v.at[j1]], add=True)

            @pl.when(c == 1)
            def _():
                pltpu.async_copy(ones_v, deg_sh.at[dst_v.at[j1]], sem_d,
                                 add=True)

            return ()

        def deg_drain(i, _):
            pltpu.make_async_copy(ones_v, deg_sh.at[dst_v.at[0]],
                                  sem_d).wait()
            return ()

        for h in range(NHALF):
            @pl.when(c == 0)
            def _():
                pltpu.sync_copy(src0_hbm.at[s].at[h], src_v)

            @pl.when(c == 1)
            def _():
                pltpu.sync_copy(src1_hbm.at[s].at[h], src_v)

            pltpu.sync_copy(dst_hbm.at[s].at[h], dst_v)
            gissue(0, rows_a, sem_a)
            lax.fori_loop(0, NPAIR_H, pair_step, ())
            lax.fori_loop(0, NPAIR_H, deg_drain, ())

        plsc.subcore_barrier()

        row0 = s * ROWS_PT

        @pl.when(jnp.logical_and(c == 0, s < NW_ROWS))
        def _():
            pltpu.sync_copy(acc_sh.at[pl.ds(row0, ROWS_PT)],
                            agg0_hbm.at[pl.ds(row0, ROWS_PT)])

        @pl.when(jnp.logical_and(c == 1, s < NW_ROWS))
        def _():
            pltpu.sync_copy(acc_sh.at[pl.ds(row0, ROWS_PT)],
                            agg1_hbm.at[pl.ds(row0, ROWS_PT)])

        @pl.when(jnp.logical_and(c == 0, s == NS - 1))
        def _():
            pltpu.sync_copy(deg_sh, deg0_hbm)

        @pl.when(jnp.logical_and(c == 1, s == NS - 1))
        def _():
            pltpu.sync_copy(deg_sh, deg1_hbm)

    return body(x2, src0_r, src1_r, dst_r, z2d, z1d, o1)


BN = 2000


def _tc_linear_body(a0_ref, a1_ref, d_ref, w_ref, b_ref, o_ref):
    recip = 1.0 / jnp.maximum(d_ref[...], 1.0)
    s0 = a0_ref[...] * recip
    s1 = a1_ref[...] * recip
    w = w_ref[...]
    acc = lax.dot_general(s0, w[:, :DH], (((1,), (1,)), ((), ())),
                          preferred_element_type=jnp.float32)
    acc += lax.dot_general(s1, w[:, DH:], (((1,), (1,)), ((), ())),
                           preferred_element_type=jnp.float32)
    o_ref[...] = acc + b_ref[...]


def _tc_linear(agg0, agg1, deg, W, b):
    grid = (N_NODES // BN,)
    return pl.pallas_call(
        _tc_linear_body,
        grid=grid,
        in_specs=[
            pl.BlockSpec((BN, DH), lambda i: (i, 0)),
            pl.BlockSpec((BN, DH), lambda i: (i, 0)),
            pl.BlockSpec((BN, 1), lambda i: (i, 0)),
            pl.BlockSpec((DIM, DIM), lambda i: (0, 0)),
            pl.BlockSpec((1, DIM), lambda i: (0, 0)),
        ],
        out_specs=pl.BlockSpec((BN, DIM), lambda i: (i, 0)),
        out_shape=jax.ShapeDtypeStruct((N_NODES, DIM), jnp.float32),
    )(agg0, agg1, deg, W, b)


def kernel(x, edge_index, W, b):
    ei = edge_index.astype(jnp.int32)
    dst = ei[0].reshape(NS, NHALF, NCHUNK_H, CHUNK)
    src2 = ei[1] * 2
    src0 = src2.reshape(NS, NHALF, NCHUNK_H, CHUNK)
    src1 = (src2 + 1).reshape(NS, NHALF, NCHUNK_H, CHUNK)
    x2 = x.reshape(2 * N_NODES, DH)
    z2d = jnp.zeros((ROWS_PT, DH), jnp.float32)
    z1d = jnp.zeros((N_NODES,), jnp.float32)
    o1 = jnp.ones((CHUNK,), jnp.float32)
    agg0, agg1, deg0, deg1 = _sc_aggregate(x2, src0, src1, dst, z2d, z1d, o1)
    deg = (deg0 + deg1).reshape(N_NODES, 1)
    return _tc_linear(agg0, agg1, deg, W, b.reshape(1, DIM))


# speedup vs baseline: 9.1472x; 1.0089x over previous
import functools

import jax
import jax.numpy as jnp
from jax import lax
from jax.experimental import pallas as pl
from jax.experimental.pallas import tpu as pltpu
from jax.experimental.pallas import tpu_sc as plsc

N_NODES = 10000
N_EDGES = 160000
DIM = 256
NC = 2
NS = 16
LANES = 16
DH = DIM // NC
EPT = N_EDGES // NS
CHUNK = 80
NCHUNK = EPT // CHUNK
NSTAGE = 5
NCHUNK_F = NCHUNK // NSTAGE
NTRI = (NCHUNK_F - 1) // 3
ROWS_PT = 1000
NW_ROWS = N_NODES // ROWS_PT


def _sc_aggregate(x2, src0_r, src1_r, dst_r, z2d, z1d, o1):
    mesh = plsc.VectorSubcoreMesh(
        core_axis_name="c", subcore_axis_name="s", num_cores=NC, num_subcores=NS
    )

    @functools.partial(
        pl.kernel,
        out_type=(
            jax.ShapeDtypeStruct((N_NODES, DH), jnp.float32),
            jax.ShapeDtypeStruct((N_NODES, DH), jnp.float32),
            jax.ShapeDtypeStruct((N_NODES,), jnp.float32),
            jax.ShapeDtypeStruct((N_NODES,), jnp.float32),
        ),
        mesh=mesh,
        scratch_types=[
            pltpu.VMEM((NCHUNK_F, CHUNK), jnp.int32),
            pltpu.VMEM((NCHUNK_F, CHUNK), jnp.int32),
            pltpu.VMEM((CHUNK, DH), jnp.float32),
            pltpu.VMEM((CHUNK, DH), jnp.float32),
            pltpu.VMEM((CHUNK, DH), jnp.float32),
            pltpu.VMEM((CHUNK,), jnp.float32),
            pltpu.VMEM_SHARED((N_NODES, DH), jnp.float32),
            pltpu.VMEM_SHARED((N_NODES,), jnp.float32),
            pltpu.SemaphoreType.DMA,
            pltpu.SemaphoreType.DMA,
            pltpu.SemaphoreType.DMA,
            pltpu.SemaphoreType.DMA,
        ],
    )
    def body(x2_hbm, src0_hbm, src1_hbm, dst_hbm, z2d_hbm, z1d_hbm, o1_hbm,
             agg0_hbm, agg1_hbm, deg0_hbm, deg1_hbm,
             src_v, dst_v, rows_a, rows_b, rows_c, ones_v, acc_sh, deg_sh,
             sem_a, sem_b, sem_c, sem_d):
        c = lax.axis_index("c")
        s = lax.axis_index("s")

        @pl.when(s < NW_ROWS)
        def _():
            pltpu.sync_copy(z2d_hbm, acc_sh.at[pl.ds(s * ROWS_PT, ROWS_PT)])

        @pl.when(s == 0)
        def _():
            pltpu.sync_copy(z1d_hbm, deg_sh)

        pltpu.sync_copy(o1_hbm, ones_v)

        plsc.subcore_barrier()

        def gissue(j, buf, sem):
            pltpu.async_copy(x2_hbm.at[src_v.at[j]], buf, sem)

        def gwait(j, buf, sem):
            pltpu.make_async_copy(x2_hbm.at[src_v.at[j]], buf, sem).wait()

        def deg_drain(i, _):
            pltpu.make_async_copy(ones_v, deg_sh.at[dst_v.at[0]],
                                  sem_d).wait()
            return ()

        for f in range(NSTAGE):
            cnt = f % 2

            @pl.when(c == 0)
            def _():
                pltpu.sync_copy(src0_hbm.at[s].at[f], src_v)

            @pl.when(c == 1)
            def _():
                pltpu.sync_copy(src1_hbm.at[s].at[f], src_v)

            pltpu.sync_copy(dst_hbm.at[s].at[f], dst_v)

            def scatter_one(j, buf):
                pltpu.sync_copy(buf, acc_sh.at[dst_v.at[j]], add=True)

                @pl.when(c == cnt)
                def _():
                    pltpu.async_copy(ones_v, deg_sh.at[dst_v.at[j]], sem_d,
                                     add=True)

            def tri_step(jt, _):
                j0 = 3 * jt
                gissue(j0 + 2, rows_c, sem_c)
                gwait(j0, rows_a, sem_a)
                scatter_one(j0, rows_a)

                @pl.when(j0 + 3 < NCHUNK_F)
                def _():
                    gissue(j0 + 3, rows_a, sem_a)

                gwait(j0 + 1, rows_b, sem_b)
                scatter_one(j0 + 1, rows_b)

                @pl.when(j0 + 4 < NCHUNK_F)
                def _():
                    gissue(j0 + 4, rows_b, sem_b)

                gwait(j0 + 2, rows_c, sem_c)
                scatter_one(j0 + 2, rows_c)
                return ()

            gissue(0, rows_a, sem_a)
            gissue(1, rows_b, sem_b)
            lax.fori_loop(0, NTRI, tri_step, ())
            gwait(NCHUNK_F - 1, rows_a, sem_a)
            scatter_one(NCHUNK_F - 1, rows_a)

            @pl.when(c == cnt)
            def _():
                lax.fori_loop(0, NCHUNK_F, deg_drain, ())

        plsc.subcore_barrier()

        row0 = s * ROWS_PT

        @pl.when(jnp.logical_and(c == 0, s < NW_ROWS))
        def _():
            pltpu.sync_copy(acc_sh.at[pl.ds(row0, ROWS_PT)],
                            agg0_hbm.at[pl.ds(row0, ROWS_PT)])

        @pl.when(jnp.logical_and(c == 1, s < NW_ROWS))
        def _():
            pltpu.sync_copy(acc_sh.at[pl.ds(row0, ROWS_PT)],
                            agg1_hbm.at[pl.ds(row0, ROWS_PT)])

        @pl.when(jnp.logical_and(c == 0, s == NS - 1))
        def _():
            pltpu.sync_copy(deg_sh, deg0_hbm)

        @pl.when(jnp.logical_and(c == 1, s == NS - 1))
        def _():
            pltpu.sync_copy(deg_sh, deg1_hbm)

    return body(x2, src0_r, src1_r, dst_r, z2d, z1d, o1)


BN = 2000


def _tc_linear_body(a0_ref, a1_ref, d_ref, w_ref, b_ref, o_ref):
    recip = 1.0 / jnp.maximum(d_ref[...], 1.0)
    s0 = a0_ref[...] * recip
    s1 = a1_ref[...] * recip
    w = w_ref[...]
    acc = lax.dot_general(s0, w[:, :DH], (((1,), (1,)), ((), ())),
                          preferred_element_type=jnp.float32)
    acc += lax.dot_general(s1, w[:, DH:], (((1,), (1,)), ((), ())),
                           preferred_element_type=jnp.float32)
    o_ref[...] = acc + b_ref[...]


def _tc_linear(agg0, agg1, deg, W, b):
    grid = (N_NODES // BN,)
    return pl.pallas_call(
        _tc_linear_body,
        grid=grid,
        in_specs=[
            pl.BlockSpec((BN, DH), lambda i: (i, 0)),
            pl.BlockSpec((BN, DH), lambda i: (i, 0)),
            pl.BlockSpec((BN, 1), lambda i: (i, 0)),
            pl.BlockSpec((DIM, DIM), lambda i: (0, 0)),
            pl.BlockSpec((1, DIM), lambda i: (0, 0)),
        ],
        out_specs=pl.BlockSpec((BN, DIM), lambda i: (i, 0)),
        out_shape=jax.ShapeDtypeStruct((N_NODES, DIM), jnp.float32),
    )(agg0, agg1, deg, W, b)


def kernel(x, edge_index, W, b):
    ei = edge_index.astype(jnp.int32)
    dst = ei[0].reshape(NS, NSTAGE, NCHUNK_F, CHUNK)
    src2 = ei[1] * 2
    src0 = src2.reshape(NS, NSTAGE, NCHUNK_F, CHUNK)
    src1 = (src2 + 1).reshape(NS, NSTAGE, NCHUNK_F, CHUNK)
    x2 = x.reshape(2 * N_NODES, DH)
    z2d = jnp.zeros((ROWS_PT, DH), jnp.float32)
    z1d = jnp.zeros((N_NODES,), jnp.float32)
    o1 = jnp.ones((CHUNK,), jnp.float32)
    agg0, agg1, deg0, deg1 = _sc_aggregate(x2, src0, src1, dst, z2d, z1d, o1)
    deg = (deg0 + deg1).reshape(N_NODES, 1)
    return _tc_linear(agg0, agg1, deg, W, b.reshape(1, DIM))


# speedup vs baseline: 9.2798x; 1.0145x over previous
import functools

import jax
import jax.numpy as jnp
from jax import lax
from jax.experimental import pallas as pl
from jax.experimental.pallas import tpu as pltpu
from jax.experimental.pallas import tpu_sc as plsc

N_NODES = 10000
N_EDGES = 160000
DIM = 256
NC = 2
NS = 16
LANES = 16
DH = DIM // NC
EPT = N_EDGES // NS
CHUNK = 80
NCHUNK = EPT // CHUNK
NSTAGE = 5
NCHUNK_F = NCHUNK // NSTAGE
NTRI = (NCHUNK_F - 1) // 3
ROWS_PT = 1000
NW_ROWS = N_NODES // ROWS_PT


def _sc_aggregate(x2, src_r, dst_r, z2d, z1d, o1):
    mesh = plsc.VectorSubcoreMesh(
        core_axis_name="c", subcore_axis_name="s", num_cores=NC, num_subcores=NS
    )

    @functools.partial(
        pl.kernel,
        out_type=(
            jax.ShapeDtypeStruct((N_NODES, DH), jnp.float32),
            jax.ShapeDtypeStruct((N_NODES, DH), jnp.float32),
            jax.ShapeDtypeStruct((N_NODES,), jnp.float32),
            jax.ShapeDtypeStruct((N_NODES,), jnp.float32),
        ),
        mesh=mesh,
        scratch_types=[
            pltpu.VMEM((NCHUNK_F, CHUNK), jnp.int32),
            pltpu.VMEM((NCHUNK_F, CHUNK), jnp.int32),
            pltpu.VMEM((CHUNK, DH), jnp.float32),
            pltpu.VMEM((CHUNK, DH), jnp.float32),
            pltpu.VMEM((CHUNK, DH), jnp.float32),
            pltpu.VMEM((CHUNK,), jnp.float32),
            pltpu.VMEM_SHARED((N_NODES, DH), jnp.float32),
            pltpu.VMEM_SHARED((N_NODES,), jnp.float32),
            pltpu.SemaphoreType.DMA,
            pltpu.SemaphoreType.DMA,
            pltpu.SemaphoreType.DMA,
            pltpu.SemaphoreType.DMA,
        ],
    )
    def body(x2_hbm, src_hbm, dst_hbm, z2d_hbm, z1d_hbm, o1_hbm,
             agg0_hbm, agg1_hbm, deg0_hbm, deg1_hbm,
             src_v, dst_v, rows_a, rows_b, rows_c, ones_v, acc_sh, deg_sh,
             sem_a, sem_b, sem_c, sem_d):
        c = lax.axis_index("c")
        s = lax.axis_index("s")

        @pl.when(s < NW_ROWS)
        def _():
            pltpu.sync_copy(z2d_hbm, acc_sh.at[pl.ds(s * ROWS_PT, ROWS_PT)])

        @pl.when(s == 0)
        def _():
            pltpu.sync_copy(z1d_hbm, deg_sh)

        pltpu.sync_copy(o1_hbm, ones_v)

        plsc.subcore_barrier()

        def gissue(j, buf, sem):
            pltpu.async_copy(x2_hbm.at[src_v.at[j]], buf, sem)

        def gwait(j, buf, sem):
            pltpu.make_async_copy(x2_hbm.at[src_v.at[j]], buf, sem).wait()

        def deg_drain(i, _):
            pltpu.make_async_copy(ones_v, deg_sh.at[dst_v.at[0]],
                                  sem_d).wait()
            return ()

        for f in range(NSTAGE):
            cnt = f % 2

            pltpu.sync_copy(src_hbm.at[s].at[f], src_v)
            pltpu.sync_copy(dst_hbm.at[s].at[f], dst_v)

            def idx_fix(r, _):
                for k in range(CHUNK // LANES):
                    sl = src_v[r, pl.ds(k * LANES, LANES)]
                    src_v[r, pl.ds(k * LANES, LANES)] = sl * 2 + c
                return ()

            lax.fori_loop(0, NCHUNK_F, idx_fix, ())

            def scatter_one(j, buf):
                pltpu.sync_copy(buf, acc_sh.at[dst_v.at[j]], add=True)

                @pl.when(c == cnt)
                def _():
                    pltpu.async_copy(ones_v, deg_sh.at[dst_v.at[j]], sem_d,
                                     add=True)

            def tri_step(jt, _):
                j0 = 3 * jt
                gissue(j0 + 2, rows_c, sem_c)
                gwait(j0, rows_a, sem_a)
                scatter_one(j0, rows_a)

                @pl.when(j0 + 3 < NCHUNK_F)
                def _():
                    gissue(j0 + 3, rows_a, sem_a)

                gwait(j0 + 1, rows_b, sem_b)
                scatter_one(j0 + 1, rows_b)

                @pl.when(j0 + 4 < NCHUNK_F)
                def _():
                    gissue(j0 + 4, rows_b, sem_b)

                gwait(j0 + 2, rows_c, sem_c)
                scatter_one(j0 + 2, rows_c)
                return ()

            gissue(0, rows_a, sem_a)
            gissue(1, rows_b, sem_b)
            lax.fori_loop(0, NTRI, tri_step, ())
            gwait(NCHUNK_F - 1, rows_a, sem_a)
            scatter_one(NCHUNK_F - 1, rows_a)

            @pl.when(c == cnt)
            def _():
                lax.fori_loop(0, NCHUNK_F, deg_drain, ())

        plsc.subcore_barrier()

        row0 = s * ROWS_PT

        @pl.when(jnp.logical_and(c == 0, s < NW_ROWS))
        def _():
            pltpu.sync_copy(acc_sh.at[pl.ds(row0, ROWS_PT)],
                            agg0_hbm.at[pl.ds(row0, ROWS_PT)])

        @pl.when(jnp.logical_and(c == 1, s < NW_ROWS))
        def _():
            pltpu.sync_copy(acc_sh.at[pl.ds(row0, ROWS_PT)],
                            agg1_hbm.at[pl.ds(row0, ROWS_PT)])

        @pl.when(jnp.logical_and(c == 0, s == NS - 1))
        def _():
            pltpu.sync_copy(deg_sh, deg0_hbm)

        @pl.when(jnp.logical_and(c == 1, s == NS - 1))
        def _():
            pltpu.sync_copy(deg_sh, deg1_hbm)

    return body(x2, src_r, dst_r, z2d, z1d, o1)


BN = 5000


def _tc_linear_body(a0_ref, a1_ref, d_ref, w_ref, b_ref, o_ref):
    recip = 1.0 / jnp.maximum(d_ref[...], 1.0)
    s0 = a0_ref[...] * recip
    s1 = a1_ref[...] * recip
    w = w_ref[...]
    acc = lax.dot_general(s0, w[:, :DH], (((1,), (1,)), ((), ())),
                          preferred_element_type=jnp.float32)
    acc += lax.dot_general(s1, w[:, DH:], (((1,), (1,)), ((), ())),
                           preferred_element_type=jnp.float32)
    o_ref[...] = acc + b_ref[...]


def _tc_linear(agg0, agg1, deg, W, b):
    grid = (N_NODES // BN,)
    return pl.pallas_call(
        _tc_linear_body,
        grid=grid,
        in_specs=[
            pl.BlockSpec((BN, DH), lambda i: (i, 0)),
            pl.BlockSpec((BN, DH), lambda i: (i, 0)),
            pl.BlockSpec((BN, 1), lambda i: (i, 0)),
            pl.BlockSpec((DIM, DIM), lambda i: (0, 0)),
            pl.BlockSpec((1, DIM), lambda i: (0, 0)),
        ],
        out_specs=pl.BlockSpec((BN, DIM), lambda i: (i, 0)),
        out_shape=jax.ShapeDtypeStruct((N_NODES, DIM), jnp.float32),
    )(agg0, agg1, deg, W, b)


def kernel(x, edge_index, W, b):
    ei = edge_index.astype(jnp.int32)
    dst = ei[0].reshape(NS, NSTAGE, NCHUNK_F, CHUNK)
    src = ei[1].reshape(NS, NSTAGE, NCHUNK_F, CHUNK)
    x2 = x.reshape(2 * N_NODES, DH)
    z2d = jnp.zeros((ROWS_PT, DH), jnp.float32)
    z1d = jnp.zeros((N_NODES,), jnp.float32)
    o1 = jnp.ones((CHUNK,), jnp.float32)
    agg0, agg1, deg0, deg1 = _sc_aggregate(x2, src, dst, z2d, z1d, o1)
    deg = (deg0 + deg1).reshape(N_NODES, 1)
    return _tc_linear(agg0, agg1, deg, W, b.reshape(1, DIM))


# speedup vs baseline: 9.8451x; 1.0609x over previous
import functools

import jax
import jax.numpy as jnp
from jax import lax
from jax.experimental import pallas as pl
from jax.experimental.pallas import tpu as pltpu
from jax.experimental.pallas import tpu_sc as plsc

N_NODES = 10000
N_EDGES = 160000
DIM = 256
NC = 2
NS = 16
LANES = 16
DH = DIM // NC
EPT = N_EDGES // NS
CHUNK = 80
NCHUNK = EPT // CHUNK
NSTAGE = 5
NCHUNK_F = NCHUNK // NSTAGE
NTRI = (NCHUNK_F - 1) // 3
ROWS_PT = 1000
NW_ROWS = N_NODES // ROWS_PT


def _sc_aggregate(x0, x1, src_r, dst_r, z2d, z1d, o1):
    mesh = plsc.VectorSubcoreMesh(
        core_axis_name="c", subcore_axis_name="s", num_cores=NC, num_subcores=NS
    )

    @functools.partial(
        pl.kernel,
        out_type=(
            jax.ShapeDtypeStruct((N_NODES, DH), jnp.float32),
            jax.ShapeDtypeStruct((N_NODES, DH), jnp.float32),
            jax.ShapeDtypeStruct((N_NODES,), jnp.float32),
            jax.ShapeDtypeStruct((N_NODES,), jnp.float32),
        ),
        mesh=mesh,
        scratch_types=[
            pltpu.VMEM((NCHUNK_F, CHUNK), jnp.int32),
            pltpu.VMEM((NCHUNK_F, CHUNK), jnp.int32),
            pltpu.VMEM((NCHUNK_F, CHUNK), jnp.int32),
            pltpu.VMEM((NCHUNK_F, CHUNK), jnp.int32),
            pltpu.VMEM((CHUNK, DH), jnp.float32),
            pltpu.VMEM((CHUNK, DH), jnp.float32),
            pltpu.VMEM((CHUNK, DH), jnp.float32),
            pltpu.VMEM((CHUNK,), jnp.float32),
            pltpu.VMEM_SHARED((N_NODES, DH), jnp.float32),
            pltpu.VMEM_SHARED((N_NODES,), jnp.float32),
            pltpu.SemaphoreType.DMA,
            pltpu.SemaphoreType.DMA,
            pltpu.SemaphoreType.DMA,
            pltpu.SemaphoreType.DMA,
            pltpu.SemaphoreType.DMA,
        ],
    )
    def body(x0_hbm, x1_hbm, src_hbm, dst_hbm, z2d_hbm, z1d_hbm, o1_hbm,
             agg0_hbm, agg1_hbm, deg0_hbm, deg1_hbm,
             src_va, dst_va, src_vb, dst_vb, rows_a, rows_b, rows_c,
             ones_v, acc_sh, deg_sh,
             sem_a, sem_b, sem_c, sem_d, sem_i):
        c = lax.axis_index("c")
        s = lax.axis_index("s")

        @pl.when(s < NW_ROWS)
        def _():
            pltpu.sync_copy(z2d_hbm, acc_sh.at[pl.ds(s * ROWS_PT, ROWS_PT)])

        @pl.when(s == 0)
        def _():
            pltpu.sync_copy(z1d_hbm, deg_sh)

        pltpu.sync_copy(o1_hbm, ones_v)

        plsc.subcore_barrier()

        def idx_load(f, sv, dv):
            pltpu.async_copy(src_hbm.at[s].at[f], sv, sem_i)
            pltpu.async_copy(dst_hbm.at[s].at[f], dv, sem_i)

        def idx_wait(f, sv, dv):
            pltpu.make_async_copy(src_hbm.at[s].at[f], sv, sem_i).wait()
            pltpu.make_async_copy(dst_hbm.at[s].at[f], dv, sem_i).wait()

        idx_load(0, src_va, dst_va)
        for f in range(NSTAGE):
            cnt = f % 2
            if f % 2 == 0:
                src_v, dst_v = src_va, dst_va
                nsrc_v, ndst_v = src_vb, dst_vb
            else:
                src_v, dst_v = src_vb, dst_vb
                nsrc_v, ndst_v = src_va, dst_va

            idx_wait(f, src_v, dst_v)
            if f + 1 < NSTAGE:
                idx_load(f + 1, nsrc_v, ndst_v)

            def gissue(j, buf, sem, sv=src_v):
                @pl.when(c == 0)
                def _():
                    pltpu.async_copy(x0_hbm.at[sv.at[j]], buf, sem)

                @pl.when(c == 1)
                def _():
                    pltpu.async_copy(x1_hbm.at[sv.at[j]], buf, sem)

            def gwait(j, buf, sem, sv=src_v):
                @pl.when(c == 0)
                def _():
                    pltpu.make_async_copy(x0_hbm.at[sv.at[j]], buf, sem).wait()

                @pl.when(c == 1)
                def _():
                    pltpu.make_async_copy(x1_hbm.at[sv.at[j]], buf, sem).wait()

            def scatter_one(j, buf, dv=dst_v, cnt=cnt):
                pltpu.sync_copy(buf, acc_sh.at[dv.at[j]], add=True)

                @pl.when(c == cnt)
                def _():
                    pltpu.async_copy(ones_v, deg_sh.at[dv.at[j]], sem_d,
                                     add=True)

            def deg_drain(i, _, dv=dst_v):
                pltpu.make_async_copy(ones_v, deg_sh.at[dv.at[0]],
                                      sem_d).wait()
                return ()

            def tri_step(jt, _):
                j0 = 3 * jt
                gissue(j0 + 2, rows_c, sem_c)
                gwait(j0, rows_a, sem_a)
                scatter_one(j0, rows_a)

                @pl.when(j0 + 3 < NCHUNK_F)
                def _():
                    gissue(j0 + 3, rows_a, sem_a)

                gwait(j0 + 1, rows_b, sem_b)
                scatter_one(j0 + 1, rows_b)

                @pl.when(j0 + 4 < NCHUNK_F)
                def _():
                    gissue(j0 + 4, rows_b, sem_b)

                gwait(j0 + 2, rows_c, sem_c)
                scatter_one(j0 + 2, rows_c)
                return ()

            gissue(0, rows_a, sem_a)
            gissue(1, rows_b, sem_b)
            lax.fori_loop(0, NTRI, tri_step, ())
            gwait(NCHUNK_F - 1, rows_a, sem_a)
            scatter_one(NCHUNK_F - 1, rows_a)

            @pl.when(c == cnt)
            def _():
                lax.fori_loop(0, NCHUNK_F, deg_drain, ())

        plsc.subcore_barrier()

        row0 = s * ROWS_PT

        @pl.when(jnp.logical_and(c == 0, s < NW_ROWS))
        def _():
            pltpu.sync_copy(acc_sh.at[pl.ds(row0, ROWS_PT)],
                            agg0_hbm.at[pl.ds(row0, ROWS_PT)])

        @pl.when(jnp.logical_and(c == 1, s < NW_ROWS))
        def _():
            pltpu.sync_copy(acc_sh.at[pl.ds(row0, ROWS_PT)],
                            agg1_hbm.at[pl.ds(row0, ROWS_PT)])

        @pl.when(jnp.logical_and(c == 0, s == NS - 1))
        def _():
            pltpu.sync_copy(deg_sh, deg0_hbm)

        @pl.when(jnp.logical_and(c == 1, s == NS - 1))
        def _():
            pltpu.sync_copy(deg_sh, deg1_hbm)

    return body(x0, x1, src_r, dst_r, z2d, z1d, o1)


BN = 5000


def _tc_linear_body(a0_ref, a1_ref, d_ref, w_ref, b_ref, o_ref):
    recip = 1.0 / jnp.maximum(d_ref[...], 1.0)
    s0 = a0_ref[...] * recip
    s1 = a1_ref[...] * recip
    w = w_ref[...]
    acc = lax.dot_general(s0, w[:, :DH], (((1,), (1,)), ((), ())),
                          preferred_element_type=jnp.float32)
    acc += lax.dot_general(s1, w[:, DH:], (((1,), (1,)), ((), ())),
                           preferred_element_type=jnp.float32)
    o_ref[...] = acc + b_ref[...]


def _tc_linear(agg0, agg1, deg, W, b):
    grid = (N_NODES // BN,)
    return pl.pallas_call(
        _tc_linear_body,
        grid=grid,
        in_specs=[
            pl.BlockSpec((BN, DH), lambda i: (i, 0)),
            pl.BlockSpec((BN, DH), lambda i: (i, 0)),
            pl.BlockSpec((BN, 1), lambda i: (i, 0)),
            pl.BlockSpec((DIM, DIM), lambda i: (0, 0)),
            pl.BlockSpec((1, DIM), lambda i: (0, 0)),
        ],
        out_specs=pl.BlockSpec((BN, DIM), lambda i: (i, 0)),
        out_shape=jax.ShapeDtypeStruct((N_NODES, DIM), jnp.float32),
    )(agg0, agg1, deg, W, b)


def kernel(x, edge_index, W, b):
    ei = edge_index.astype(jnp.int32)
    dst = ei[0].reshape(NS, NSTAGE, NCHUNK_F, CHUNK)
    src = ei[1].reshape(NS, NSTAGE, NCHUNK_F, CHUNK)
    x0 = x[:, :DH]
    x1 = x[:, DH:]
    z2d = jnp.zeros((ROWS_PT, DH), jnp.float32)
    z1d = jnp.zeros((N_NODES,), jnp.float32)
    o1 = jnp.ones((CHUNK,), jnp.float32)
    agg0, agg1, deg0, deg1 = _sc_aggregate(x0, x1, src, dst, z2d, z1d, o1)
    deg = (deg0 + deg1).reshape(N_NODES, 1)
    return _tc_linear(agg0, agg1, deg, W, b.reshape(1, DIM))
